# baseline (speedup 1.0000x reference)
import dataclasses

import jax
import jax.numpy as jnp
from jax import lax
from jax.experimental import pallas as pl
from jax.experimental.pallas import tpu as pltpu
from jax.experimental.pallas import tpu_sc as plsc

M = 100000
B = 16384
D = 128

RPT = 3200
W = 160
NW = RPT // W
NBUF = 4
CH = 128
LANES = 16
NS = B // LANES

_GATHER_DNUMS = lax.GatherDimensionNumbers(
    offset_dims=(), collapsed_slice_dims=(0,), start_index_map=(0,))


def _bcast16(vals, pos):
    idx = jnp.full((LANES,), pos, jnp.int32)
    return lax.gather(vals, idx[:, None], _GATHER_DNUMS, slice_sizes=(1,),
                      mode=lax.GatherScatterMode.PROMISE_IN_BOUNDS)


def _scatter_add_kernel(var_hbm, value_hbm, idx_hbm, samp_hbm, alpha_hbm,
                        out_hbm, b0, b1, b2, b3, samp_v, vc0, vc1, ic0, ic1,
                        alpha_v, isems, wsems, vsems, csems):
    c = lax.axis_index("c")
    s = lax.axis_index("s")
    tg = s * 2 + c

    lane_iota = lax.iota(jnp.int32, LANES)
    start = pl.multiple_of(
        jnp.minimum(tg * RPT, jnp.int32(M - RPT)), 8)

    bufs = (b0, b1, b2, b3)
    vcs = (vc0, vc1)
    ics = (ic0, ic1)

    def win_rows(k):
        return pl.multiple_of(start + k * W, 8)

    pltpu.async_copy(var_hbm.at[pl.ds(win_rows(0), W), :], bufs[0],
                     isems.at[0])
    pltpu.async_copy(var_hbm.at[pl.ds(win_rows(1), W), :], bufs[1],
                     isems.at[1])

    pltpu.sync_copy(samp_hbm, samp_v)
    pltpu.sync_copy(alpha_hbm, alpha_v)
    avec = alpha_v[...]
    alpha_s = lax.reduce_max(avec, axes=(0,))

    def search16(bounds):
        lo = jnp.zeros((LANES,), jnp.int32)
        hi = jnp.full((LANES,), NS, jnp.int32)
        for _ in range(11):
            live = lo < hi
            mid = lax.div(lo + hi, jnp.int32(2))
            vals = plsc.load_gather(
                samp_v, [jnp.minimum(mid, jnp.int32(NS - 1))])
            pred = live & (vals < bounds)
            lo = jnp.where(pred, mid + 1, lo)
            hi = jnp.where(live & jnp.logical_not(vals < bounds), mid, hi)
        return lo

    q0 = search16(start + lane_iota * W)
    q1 = search16(start + (LANES + lane_iota) * W)
    jlo0 = jnp.maximum(q0 - 1, 0) * LANES
    jlo1 = jnp.maximum(q1 - 1, 0) * LANES
    jhi0 = q0 * LANES
    jhi1 = q1 * LANES

    def extract(vec0, vec1, k):
        e0 = lax.reduce_max(
            jnp.where(lane_iota == k, vec0, jnp.int32(0)), axes=(0,))
        e1 = lax.reduce_max(
            jnp.where(lane_iota == k - LANES, vec1, jnp.int32(0)), axes=(0,))
        return jnp.where(k < LANES, e0, e1)

    cols = [m * LANES + lane_iota for m in range(D // LANES)]

    def chunk_start(j):
        return pl.multiple_of(
            jnp.minimum(lax.div(j, jnp.int32(CH)) * CH, jnp.int32(B - CH)), 8)

    def prefetch_chunk(j, slot):
        kk = chunk_start(j)
        pltpu.async_copy(value_hbm.at[pl.ds(kk, CH), :], vcs[slot],
                         vsems.at[slot])
        pltpu.async_copy(idx_hbm.at[pl.ds(kk, CH)], ics[slot],
                         csems.at[slot])

    jb0 = extract(jlo0, jlo1, jnp.int32(0))
    prefetch_chunk(jb0, 0)

    def process_chunk(buf, vc, ic, kk, w, jlo, jhi):
        @pl.when(alpha_s != 1.0)
        def _():
            @pl.loop(0, CH)
            def _(r):
                for m in range(D // LANES):
                    sl = pl.ds(m * LANES, LANES)
                    vc[r, sl] = vc[r, sl] * avec

        lo_c = jnp.maximum(jlo, kk)
        hi_c = jnp.minimum(jhi, kk + CH)

        def j_body(j, _):
            jr = j - kk
            grp = lax.div(jr, jnp.int32(LANES)) * LANES
            iv = ic[pl.ds(grp, LANES)]
            rowv = _bcast16(iv, jr - grp) - w
            mask = (rowv >= 0) & (rowv < W)
            rowc = jnp.clip(rowv, 0, W - 1)
            for m in range(D // LANES):
                x = vc[jr, pl.ds(m * LANES, LANES)]
                plsc.addupdate_scatter(buf, [rowc, cols[m]], x, mask=mask)
            return 0

        lax.fori_loop(lo_c, hi_c, j_body, 0)

    @pl.loop(0, NW // NBUF)
    def _(g):
        for bslot in range(NBUF):
            k = g * NBUF + bslot
            buf = bufs[bslot]
            vslot = bslot % 2
            vc = vcs[vslot]
            ic = ics[vslot]
            w = win_rows(k)

            @pl.when(k + 2 < NW)
            def _():
                nslot = (bslot + 2) % NBUF

                @pl.when(k >= 2)
                def _():
                    pltpu.make_async_copy(
                        bufs[nslot],
                        out_hbm.at[pl.ds(win_rows(k - 2), W), :],
                        wsems.at[nslot]).wait()

                pltpu.async_copy(
                    var_hbm.at[pl.ds(win_rows(k + 2), W), :],
                    bufs[nslot], isems.at[nslot])

            jlo = extract(jlo0, jlo1, k)
            jhi = extract(jhi0, jhi1, k + 1)
            kk0 = chunk_start(jlo)

            @pl.when(k + 1 < NW)
            def _():
                jlo_n = extract(jlo0, jlo1, k + 1)
                prefetch_chunk(jlo_n, (vslot + 1) % 2)

            pltpu.make_async_copy(
                var_hbm.at[pl.ds(w, W), :], buf, isems.at[bslot]).wait()
            pltpu.make_async_copy(
                value_hbm.at[pl.ds(kk0, CH), :], vc, vsems.at[vslot]).wait()
            pltpu.make_async_copy(
                idx_hbm.at[pl.ds(kk0, CH)], ic, csems.at[vslot]).wait()

            @pl.when(jlo < jhi)
            def _():
                process_chunk(buf, vc, ic, kk0, w, jlo, jhi)

                def extra_body(kk):
                    kk = pl.multiple_of(kk, 8)
                    pltpu.sync_copy(value_hbm.at[pl.ds(kk, CH), :], vc)
                    pltpu.sync_copy(idx_hbm.at[pl.ds(kk, CH)], ic)
                    process_chunk(buf, vc, ic, kk, w, jlo, jhi)
                    return kk + CH

                lax.while_loop(lambda kk: kk < jhi, extra_body, kk0 + CH)

            pltpu.async_copy(buf, out_hbm.at[pl.ds(w, W), :],
                             wsems.at[bslot])

    for k in range(NW - NBUF, NW):
        pltpu.make_async_copy(
            bufs[k % NBUF],
            out_hbm.at[pl.ds(win_rows(k), W), :],
            wsems.at[k % NBUF]).wait()


def kernel(var, value, sorted_indices, pos, alpha):
    del pos
    idx32 = sorted_indices.astype(jnp.int32)
    sampled = idx32.reshape(NS, LANES)[:, 0]
    alpha_vec = jnp.broadcast_to(
        jnp.asarray(alpha, jnp.float32).reshape(1), (LANES,))

    cp = pltpu.CompilerParams()
    if "needs_layout_passes" in pltpu.CompilerParams.__dataclass_fields__:
        cp = dataclasses.replace(cp, needs_layout_passes=False)

    mesh = plsc.VectorSubcoreMesh(core_axis_name="c", subcore_axis_name="s")
    run = pl.kernel(
        _scatter_add_kernel,
        out_type=jax.ShapeDtypeStruct((M, D), jnp.float32),
        mesh=mesh,
        scratch_types=[
            pltpu.VMEM((W, D), jnp.float32),
            pltpu.VMEM((W, D), jnp.float32),
            pltpu.VMEM((W, D), jnp.float32),
            pltpu.VMEM((W, D), jnp.float32),
            pltpu.VMEM((NS,), jnp.int32),
            pltpu.VMEM((CH, D), jnp.float32),
            pltpu.VMEM((CH, D), jnp.float32),
            pltpu.VMEM((CH,), jnp.int32),
            pltpu.VMEM((CH,), jnp.int32),
            pltpu.VMEM((LANES,), jnp.float32),
            pltpu.SemaphoreType.DMA((NBUF,)),
            pltpu.SemaphoreType.DMA((NBUF,)),
            pltpu.SemaphoreType.DMA((2,)),
            pltpu.SemaphoreType.DMA((2,)),
        ],
        compiler_params=cp,
    )
    return run(var, value, idx32, sampled, alpha_vec)


# speedup vs baseline: 1.0111x; 1.0111x over previous
import dataclasses

import jax
import jax.numpy as jnp
from jax import lax
from jax.experimental import pallas as pl
from jax.experimental.pallas import tpu as pltpu
from jax.experimental.pallas import tpu_sc as plsc

M = 100000
B = 16384
D = 128

RPT = 3200
W = 160
NW = RPT // W
NBUF = 4
CH = 64
LANES = 16

_GATHER_DNUMS = lax.GatherDimensionNumbers(
    offset_dims=(), collapsed_slice_dims=(0,), start_index_map=(0,))


def _bcast16(vals, pos):
    idx = jnp.full((LANES,), pos, jnp.int32)
    return lax.gather(vals, idx[:, None], _GATHER_DNUMS, slice_sizes=(1,),
                      mode=lax.GatherScatterMode.PROMISE_IN_BOUNDS)


def _scatter_add_kernel(var_hbm, value_hbm, idx_hbm, alpha_hbm, out_hbm,
                        b0, b1, b2, b3, idx_v, vc0, vc1, alpha_v,
                        isems, wsems, vsems):
    c = lax.axis_index("c")
    s = lax.axis_index("s")
    tg = s * 2 + c

    lane_iota = lax.iota(jnp.int32, LANES)
    start = pl.multiple_of(
        jnp.minimum(tg * RPT, jnp.int32(M - RPT)), 8)

    pltpu.async_copy(
        var_hbm.at[pl.ds(pl.multiple_of(start, 8), W), :], b0, isems.at[0])
    pltpu.async_copy(
        var_hbm.at[pl.ds(pl.multiple_of(start + W, 8), W), :], b1,
        isems.at[1])

    pltpu.sync_copy(idx_hbm, idx_v)
    pltpu.sync_copy(alpha_hbm, alpha_v)
    avec = alpha_v[...]
    alpha_s = lax.reduce_max(avec, axes=(0,))

    def search16(bounds):
        lo = jnp.zeros((LANES,), jnp.int32)
        hi = jnp.full((LANES,), B, jnp.int32)
        for _ in range(15):
            live = lo < hi
            mid = lax.div(lo + hi, jnp.int32(2))
            vals = plsc.load_gather(
                idx_v, [jnp.minimum(mid, jnp.int32(B - 1))])
            pred = live & (vals < bounds)
            lo = jnp.where(pred, mid + 1, lo)
            hi = jnp.where(live & jnp.logical_not(vals < bounds), mid, hi)
        return lo

    q0 = search16(start + lane_iota * W)
    q1 = search16(start + (LANES + lane_iota) * W)

    def extract(k):
        e0 = lax.reduce_max(
            jnp.where(lane_iota == k, q0, jnp.int32(0)), axes=(0,))
        e1 = lax.reduce_max(
            jnp.where(lane_iota == k - LANES, q1, jnp.int32(0)), axes=(0,))
        return jnp.where(k < LANES, e0, e1)

    bufs = (b0, b1, b2, b3)
    vcs = (vc0, vc1)
    cols = [m * LANES + lane_iota for m in range(D // LANES)]

    def win_rows(k):
        return pl.multiple_of(start + k * W, 8)

    def chunk_start(j):
        return pl.multiple_of(
            jnp.minimum(lax.div(j, jnp.int32(CH)) * CH, jnp.int32(B - CH)), 8)

    jb0 = extract(jnp.int32(0))
    pltpu.async_copy(value_hbm.at[pl.ds(chunk_start(jb0), CH), :],
                     vcs[0], vsems.at[0])

    def process_chunk(buf, vc, kk, w, jlo, jhi):
        @pl.when(alpha_s != 1.0)
        def _():
            @pl.loop(0, CH)
            def _(r):
                for m in range(D // LANES):
                    sl = pl.ds(m * LANES, LANES)
                    vc[r, sl] = vc[r, sl] * avec

        lo_c = jnp.maximum(jlo, kk)
        hi_c = jnp.minimum(jhi, kk + CH)
        g_lo = lax.div(lo_c - kk, jnp.int32(LANES))
        g_hi = lax.div(hi_c - kk + jnp.int32(LANES - 1), jnp.int32(LANES))

        def group_body(gi, _):
            gbase = gi * LANES
            iv = idx_v[pl.ds(kk + gbase, LANES)]
            rowv = iv - w
            jpos = kk + gbase + lane_iota
            maskv = ((jpos >= lo_c) & (jpos < hi_c)
                     & (rowv >= 0) & (rowv < W))
            rowcv = jnp.clip(rowv, 0, W - 1)
            mi = maskv.astype(jnp.int32)
            for u in range(LANES):
                rowu = _bcast16(rowcv, u)
                mu = _bcast16(mi, u) != 0
                for m in range(D // LANES):
                    x = vc[gbase + u, pl.ds(m * LANES, LANES)]
                    plsc.addupdate_scatter(buf, [rowu, cols[m]], x, mask=mu)
            return 0

        lax.fori_loop(g_lo, g_hi, group_body, 0)

    @pl.loop(0, NW // NBUF)
    def _(g):
        for bslot in range(NBUF):
            k = g * NBUF + bslot
            buf = bufs[bslot]
            vslot = bslot % 2
            vc = vcs[vslot]
            w = win_rows(k)

            @pl.when(k + 2 < NW)
            def _():
                nslot = (bslot + 2) % NBUF

                @pl.when(k >= 2)
                def _():
                    pltpu.make_async_copy(
                        bufs[nslot],
                        out_hbm.at[pl.ds(win_rows(k - 2), W), :],
                        wsems.at[nslot]).wait()

                pltpu.async_copy(
                    var_hbm.at[pl.ds(win_rows(k + 2), W), :],
                    bufs[nslot], isems.at[nslot])

            jlo = extract(k)
            jhi = extract(k + 1)
            kk0 = chunk_start(jlo)

            @pl.when(k + 1 < NW)
            def _():
                pltpu.async_copy(
                    value_hbm.at[pl.ds(chunk_start(jhi), CH), :],
                    vcs[(vslot + 1) % 2], vsems.at[(vslot + 1) % 2])

            pltpu.make_async_copy(
                var_hbm.at[pl.ds(w, W), :], buf, isems.at[bslot]).wait()
            pltpu.make_async_copy(
                value_hbm.at[pl.ds(kk0, CH), :], vc, vsems.at[vslot]).wait()

            @pl.when(jlo < jhi)
            def _():
                process_chunk(buf, vc, kk0, w, jlo, jhi)

                def extra_body(kk):
                    kk = pl.multiple_of(kk, 8)
                    pltpu.sync_copy(value_hbm.at[pl.ds(kk, CH), :], vc)
                    process_chunk(buf, vc, kk, w, jlo, jhi)
                    return kk + CH

                lax.while_loop(lambda kk: kk < jhi, extra_body, kk0 + CH)

            pltpu.async_copy(buf, out_hbm.at[pl.ds(w, W), :],
                             wsems.at[bslot])

    for k in range(NW - NBUF, NW):
        pltpu.make_async_copy(
            bufs[k % NBUF],
            out_hbm.at[pl.ds(win_rows(k), W), :],
            wsems.at[k % NBUF]).wait()


def kernel(var, value, sorted_indices, pos, alpha):
    del pos
    alpha_vec = jnp.broadcast_to(
        jnp.asarray(alpha, jnp.float32).reshape(1), (LANES,))

    cp = pltpu.CompilerParams()
    if "needs_layout_passes" in pltpu.CompilerParams.__dataclass_fields__:
        cp = dataclasses.replace(cp, needs_layout_passes=False)

    mesh = plsc.VectorSubcoreMesh(core_axis_name="c", subcore_axis_name="s")
    run = pl.kernel(
        _scatter_add_kernel,
        out_type=jax.ShapeDtypeStruct((M, D), jnp.float32),
        mesh=mesh,
        scratch_types=[
            pltpu.VMEM((W, D), jnp.float32),
            pltpu.VMEM((W, D), jnp.float32),
            pltpu.VMEM((W, D), jnp.float32),
            pltpu.VMEM((W, D), jnp.float32),
            pltpu.VMEM((B,), jnp.int32),
            pltpu.VMEM((CH, D), jnp.float32),
            pltpu.VMEM((CH, D), jnp.float32),
            pltpu.VMEM((LANES,), jnp.float32),
            pltpu.SemaphoreType.DMA((NBUF,)),
            pltpu.SemaphoreType.DMA((NBUF,)),
            pltpu.SemaphoreType.DMA((2,)),
        ],
        compiler_params=cp,
    )
    return run(var, value, sorted_indices.astype(jnp.int32), alpha_vec)


# speedup vs baseline: 1.0811x; 1.0692x over previous
import dataclasses

import jax
import jax.numpy as jnp
from jax import lax
from jax.experimental import pallas as pl
from jax.experimental.pallas import tpu as pltpu
from jax.experimental.pallas import tpu_sc as plsc

M = 100000
B = 16384
D = 128

RPT = 3200
W = 160
NW = RPT // W
NBUF = 4
CH = 64
LANES = 16

_GATHER_DNUMS = lax.GatherDimensionNumbers(
    offset_dims=(), collapsed_slice_dims=(0,), start_index_map=(0,))


def _bcast16(vals, pos):
    idx = jnp.full((LANES,), pos, jnp.int32)
    return lax.gather(vals, idx[:, None], _GATHER_DNUMS, slice_sizes=(1,),
                      mode=lax.GatherScatterMode.PROMISE_IN_BOUNDS)


def _scatter_add_kernel(var_hbm, value_hbm, idx_hbm, alpha_hbm, out_hbm,
                        b0, b1, b2, b3, idx_v, vc0, vc1, alpha_v,
                        isems, wsems, vsems):
    c = lax.axis_index("c")
    s = lax.axis_index("s")
    tg = s * 2 + c

    lane_iota = lax.iota(jnp.int32, LANES)
    start = pl.multiple_of(
        jnp.minimum(tg * RPT, jnp.int32(M - RPT)), 8)

    pltpu.async_copy(
        var_hbm.at[pl.ds(pl.multiple_of(start, 8), W), :], b0, isems.at[0])
    pltpu.async_copy(
        var_hbm.at[pl.ds(pl.multiple_of(start + W, 8), W), :], b1,
        isems.at[1])

    pltpu.sync_copy(idx_hbm, idx_v)
    pltpu.sync_copy(alpha_hbm, alpha_v)
    avec = alpha_v[...]
    alpha_s = lax.reduce_max(avec, axes=(0,))

    def search16(bounds):
        lo = jnp.zeros((LANES,), jnp.int32)
        hi = jnp.full((LANES,), B, jnp.int32)
        for _ in range(15):
            live = lo < hi
            mid = lax.div(lo + hi, jnp.int32(2))
            vals = plsc.load_gather(
                idx_v, [jnp.minimum(mid, jnp.int32(B - 1))])
            pred = live & (vals < bounds)
            lo = jnp.where(pred, mid + 1, lo)
            hi = jnp.where(live & jnp.logical_not(vals < bounds), mid, hi)
        return lo

    q0 = search16(start + lane_iota * W)
    q1 = search16(start + (LANES + lane_iota) * W)

    def extract(k):
        e0 = lax.reduce_max(
            jnp.where(lane_iota == k, q0, jnp.int32(0)), axes=(0,))
        e1 = lax.reduce_max(
            jnp.where(lane_iota == k - LANES, q1, jnp.int32(0)), axes=(0,))
        return jnp.where(k < LANES, e0, e1)

    bufs = (b0, b1, b2, b3)
    vcs = (vc0, vc1)
    cols = [m * LANES + lane_iota for m in range(D // LANES)]

    def win_rows(k):
        return pl.multiple_of(start + k * W, 8)

    def chunk_start(j):
        return pl.multiple_of(
            jnp.minimum(lax.div(j, jnp.int32(CH)) * CH, jnp.int32(B - CH)), 8)

    jb0 = extract(jnp.int32(0))
    pltpu.async_copy(value_hbm.at[pl.ds(chunk_start(jb0), CH), :],
                     vcs[0], vsems.at[0])

    def process_chunk(buf, vc, kk, w, jlo, jhi):
        @pl.when(alpha_s != 1.0)
        def _():
            @pl.loop(0, CH)
            def _(r):
                for m in range(D // LANES):
                    sl = pl.ds(m * LANES, LANES)
                    vc[r, sl] = vc[r, sl] * avec

        lo_c = jnp.maximum(jlo, kk)
        hi_c = jnp.minimum(jhi, kk + CH)

        def j_body(j, _):
            jr = j - kk
            grp = lax.div(jr, jnp.int32(LANES)) * LANES
            iv = idx_v[pl.ds(kk + grp, LANES)]
            rowv = _bcast16(iv, jr - grp) - w
            mask = (rowv >= 0) & (rowv < W)
            rowc = jnp.clip(rowv, 0, W - 1)
            for m in range(D // LANES):
                x = vc[jr, pl.ds(m * LANES, LANES)]
                plsc.addupdate_scatter(buf, [rowc, cols[m]], x, mask=mask)
            return 0

        lax.fori_loop(lo_c, hi_c, j_body, 0)

    @pl.loop(0, NW // NBUF)
    def _(g):
        for bslot in range(NBUF):
            k = g * NBUF + bslot
            buf = bufs[bslot]
            vslot = bslot % 2
            vc = vcs[vslot]
            w = win_rows(k)

            @pl.when(k + 2 < NW)
            def _():
                nslot = (bslot + 2) % NBUF

                @pl.when(k >= 2)
                def _():
                    pltpu.make_async_copy(
                        bufs[nslot],
                        out_hbm.at[pl.ds(win_rows(k - 2), W), :],
                        wsems.at[nslot]).wait()

                pltpu.async_copy(
                    var_hbm.at[pl.ds(win_rows(k + 2), W), :],
                    bufs[nslot], isems.at[nslot])

            jlo = extract(k)
            jhi = extract(k + 1)
            kk0 = chunk_start(jlo)

            @pl.when(k + 1 < NW)
            def _():
                pltpu.async_copy(
                    value_hbm.at[pl.ds(chunk_start(jhi), CH), :],
                    vcs[(vslot + 1) % 2], vsems.at[(vslot + 1) % 2])

            pltpu.make_async_copy(
                var_hbm.at[pl.ds(w, W), :], buf, isems.at[bslot]).wait()
            pltpu.make_async_copy(
                value_hbm.at[pl.ds(kk0, CH), :], vc, vsems.at[vslot]).wait()

            @pl.when(jlo < jhi)
            def _():
                process_chunk(buf, vc, kk0, w, jlo, jhi)

                def extra_body(kk):
                    kk = pl.multiple_of(kk, 8)
                    pltpu.sync_copy(value_hbm.at[pl.ds(kk, CH), :], vc)
                    process_chunk(buf, vc, kk, w, jlo, jhi)
                    return kk + CH

                lax.while_loop(lambda kk: kk < jhi, extra_body, kk0 + CH)

            pltpu.async_copy(buf, out_hbm.at[pl.ds(w, W), :],
                             wsems.at[bslot])

    for k in range(NW - NBUF, NW):
        pltpu.make_async_copy(
            bufs[k % NBUF],
            out_hbm.at[pl.ds(win_rows(k), W), :],
            wsems.at[k % NBUF]).wait()


def kernel(var, value, sorted_indices, pos, alpha):
    del pos
    alpha_vec = jnp.broadcast_to(
        jnp.asarray(alpha, jnp.float32).reshape(1), (LANES,))

    cp = pltpu.CompilerParams()
    if "needs_layout_passes" in pltpu.CompilerParams.__dataclass_fields__:
        cp = dataclasses.replace(cp, needs_layout_passes=False)

    mesh = plsc.VectorSubcoreMesh(core_axis_name="c", subcore_axis_name="s")
    run = pl.kernel(
        _scatter_add_kernel,
        out_type=jax.ShapeDtypeStruct((M, D), jnp.float32),
        mesh=mesh,
        scratch_types=[
            pltpu.VMEM((W, D), jnp.float32),
            pltpu.VMEM((W, D), jnp.float32),
            pltpu.VMEM((W, D), jnp.float32),
            pltpu.VMEM((W, D), jnp.float32),
            pltpu.VMEM((B,), jnp.int32),
            pltpu.VMEM((CH, D), jnp.float32),
            pltpu.VMEM((CH, D), jnp.float32),
            pltpu.VMEM((LANES,), jnp.float32),
            pltpu.SemaphoreType.DMA((NBUF,)),
            pltpu.SemaphoreType.DMA((NBUF,)),
            pltpu.SemaphoreType.DMA((2,)),
        ],
        compiler_params=cp,
    )
    return run(var, value, sorted_indices.astype(jnp.int32), alpha_vec)


# speedup vs baseline: 1.0913x; 1.0094x over previous
import dataclasses

import jax
import jax.numpy as jnp
from jax import lax
from jax.experimental import pallas as pl
from jax.experimental.pallas import tpu as pltpu
from jax.experimental.pallas import tpu_sc as plsc

M = 100000
B = 16384
D = 128

RPT = 3200
W = 160
NW = RPT // W
NBUF = 4
CH = 64
LANES = 16

_GATHER_DNUMS = lax.GatherDimensionNumbers(
    offset_dims=(), collapsed_slice_dims=(0,), start_index_map=(0,))


def _bcast16(vals, pos):
    idx = jnp.full((LANES,), pos, jnp.int32)
    return lax.gather(vals, idx[:, None], _GATHER_DNUMS, slice_sizes=(1,),
                      mode=lax.GatherScatterMode.PROMISE_IN_BOUNDS)


def _scatter_add_kernel(var_hbm, value_hbm, idx_hbm, alpha_hbm, out_hbm,
                        b0, b1, b2, b3, idx_v, vc0, vc1, alpha_v,
                        isems, wsems, vsems):
    c = lax.axis_index("c")
    s = lax.axis_index("s")
    tg = s * 2 + c

    pltpu.sync_copy(idx_hbm, idx_v)
    pltpu.sync_copy(alpha_hbm, alpha_v)
    avec = alpha_v[...]
    alpha_s = lax.reduce_max(avec, axes=(0,))

    lane_iota = lax.iota(jnp.int32, LANES)
    start = pl.multiple_of(
        jnp.minimum(tg * RPT, jnp.int32(M - RPT)), 8)

    def search16(bounds):
        lo = jnp.zeros((LANES,), jnp.int32)
        hi = jnp.full((LANES,), B, jnp.int32)
        for _ in range(15):
            live = lo < hi
            mid = lax.div(lo + hi, jnp.int32(2))
            vals = plsc.load_gather(
                idx_v, [jnp.minimum(mid, jnp.int32(B - 1))])
            pred = live & (vals < bounds)
            lo = jnp.where(pred, mid + 1, lo)
            hi = jnp.where(live & jnp.logical_not(vals < bounds), mid, hi)
        return lo

    q0 = search16(start + lane_iota * W)
    q1 = search16(start + (LANES + lane_iota) * W)

    def extract(k):
        e0 = lax.reduce_max(
            jnp.where(lane_iota == k, q0, jnp.int32(0)), axes=(0,))
        e1 = lax.reduce_max(
            jnp.where(lane_iota == k - LANES, q1, jnp.int32(0)), axes=(0,))
        return jnp.where(k < LANES, e0, e1)

    bufs = (b0, b1, b2, b3)
    vcs = (vc0, vc1)
    cols = [m * LANES + lane_iota for m in range(D // LANES)]

    def win_rows(k):
        return pl.multiple_of(start + k * W, 8)

    def chunk_start(j):
        return pl.multiple_of(
            jnp.minimum(lax.div(j, jnp.int32(CH)) * CH, jnp.int32(B - CH)), 8)

    pltpu.async_copy(var_hbm.at[pl.ds(win_rows(0), W), :], bufs[0],
                     isems.at[0])
    pltpu.async_copy(var_hbm.at[pl.ds(win_rows(1), W), :], bufs[1],
                     isems.at[1])
    jb0 = extract(jnp.int32(0))
    pltpu.async_copy(value_hbm.at[pl.ds(chunk_start(jb0), CH), :],
                     vcs[0], vsems.at[0])

    def process_chunk(buf, vc, kk, w, jlo, jhi):
        @pl.when(alpha_s != 1.0)
        def _():
            @pl.loop(0, CH)
            def _(r):
                for m in range(D // LANES):
                    sl = pl.ds(m * LANES, LANES)
                    vc[r, sl] = vc[r, sl] * avec

        lo_c = jnp.maximum(jlo, kk)
        hi_c = jnp.minimum(jhi, kk + CH)

        def j_body(j, _):
            jr = j - kk
            grp = lax.div(jr, jnp.int32(LANES)) * LANES
            iv = idx_v[pl.ds(kk + grp, LANES)]
            rowv = _bcast16(iv, jr - grp) - w
            mask = (rowv >= 0) & (rowv < W)
            rowc = jnp.clip(rowv, 0, W - 1)
            for m in range(D // LANES):
                x = vc[jr, pl.ds(m * LANES, LANES)]
                plsc.addupdate_scatter(buf, [rowc, cols[m]], x, mask=mask)
            return 0

        lax.fori_loop(lo_c, hi_c, j_body, 0)

    @pl.loop(0, NW // NBUF)
    def _(g):
        for bslot in range(NBUF):
            k = g * NBUF + bslot
            buf = bufs[bslot]
            vslot = bslot % 2
            vc = vcs[vslot]
            w = win_rows(k)

            @pl.when(k + 2 < NW)
            def _():
                nslot = (bslot + 2) % NBUF

                @pl.when(k >= 2)
                def _():
                    pltpu.make_async_copy(
                        bufs[nslot],
                        out_hbm.at[pl.ds(win_rows(k - 2), W), :],
                        wsems.at[nslot]).wait()

                pltpu.async_copy(
                    var_hbm.at[pl.ds(win_rows(k + 2), W), :],
                    bufs[nslot], isems.at[nslot])

            jlo = extract(k)
            jhi = extract(k + 1)
            kk0 = chunk_start(jlo)

            @pl.when(k + 1 < NW)
            def _():
                pltpu.async_copy(
                    value_hbm.at[pl.ds(chunk_start(jhi), CH), :],
                    vcs[(vslot + 1) % 2], vsems.at[(vslot + 1) % 2])

            pltpu.make_async_copy(
                var_hbm.at[pl.ds(w, W), :], buf, isems.at[bslot]).wait()
            pltpu.make_async_copy(
                value_hbm.at[pl.ds(kk0, CH), :], vc, vsems.at[vslot]).wait()

            @pl.when(jlo < jhi)
            def _():
                process_chunk(buf, vc, kk0, w, jlo, jhi)

                def extra_body(kk):
                    kk = pl.multiple_of(kk, 8)
                    pltpu.sync_copy(value_hbm.at[pl.ds(kk, CH), :], vc)
                    process_chunk(buf, vc, kk, w, jlo, jhi)
                    return kk + CH

                lax.while_loop(lambda kk: kk < jhi, extra_body, kk0 + CH)

            pltpu.async_copy(buf, out_hbm.at[pl.ds(w, W), :],
                             wsems.at[bslot])

    for k in range(NW - NBUF, NW):
        pltpu.make_async_copy(
            bufs[k % NBUF],
            out_hbm.at[pl.ds(win_rows(k), W), :],
            wsems.at[k % NBUF]).wait()


def kernel(var, value, sorted_indices, pos, alpha):
    del pos
    alpha_vec = jnp.broadcast_to(
        jnp.asarray(alpha, jnp.float32).reshape(1), (LANES,))

    cp = pltpu.CompilerParams()
    if "needs_layout_passes" in pltpu.CompilerParams.__dataclass_fields__:
        cp = dataclasses.replace(cp, needs_layout_passes=False)

    mesh = plsc.VectorSubcoreMesh(core_axis_name="c", subcore_axis_name="s")
    run = pl.kernel(
        _scatter_add_kernel,
        out_type=jax.ShapeDtypeStruct((M, D), jnp.float32),
        mesh=mesh,
        scratch_types=[
            pltpu.VMEM((W, D), jnp.float32),
            pltpu.VMEM((W, D), jnp.float32),
            pltpu.VMEM((W, D), jnp.float32),
            pltpu.VMEM((W, D), jnp.float32),
            pltpu.VMEM((B,), jnp.int32),
            pltpu.VMEM((CH, D), jnp.float32),
            pltpu.VMEM((CH, D), jnp.float32),
            pltpu.VMEM((LANES,), jnp.float32),
            pltpu.SemaphoreType.DMA((NBUF,)),
            pltpu.SemaphoreType.DMA((NBUF,)),
            pltpu.SemaphoreType.DMA((2,)),
        ],
        compiler_params=cp,
    )
    return run(var, value, sorted_indices.astype(jnp.int32), alpha_vec)
